# baseline (speedup 1.0000x reference)
import functools

import jax
import jax.numpy as jnp
from jax import lax
from jax.experimental import pallas as pl
from jax.experimental.pallas import tpu as pltpu
from jax.experimental.pallas import tpu_sc as plsc

_CH = 128
_ROWPAD = 128


def _sc_segment_sum(table, ei_real, ei_pad):
    n_pad, d = table.shape
    n_chunks_real = ei_real.shape[1]
    pad_chunks = ei_pad.shape[1]
    n_chunks = n_chunks_real + pad_chunks
    info = plsc.get_sparse_core_info()
    nc, ns = info.num_cores, info.num_subcores
    nw = nc * ns
    cpt = n_chunks // nw
    assert cpt * nw == n_chunks and cpt % 4 == 0
    assert pad_chunks < cpt
    split = n_chunks_real - (nw - 1) * cpt
    zrow0 = n_pad - _ROWPAD
    rpt = ((n_pad + ns - 1) // ns + 7) // 8 * 8
    rpt_last = n_pad - rpt * (ns - 1)
    assert rpt_last > 0 and rpt_last % 8 == 0

    mesh = plsc.VectorSubcoreMesh(core_axis_name="c", subcore_axis_name="s")

    @functools.partial(
        pl.kernel,
        out_type=jax.ShapeDtypeStruct((nc, n_pad, 128), jnp.float32),
        mesh=mesh,
        scratch_types=[
            pltpu.VMEM((2, cpt, _CH), jnp.int32),
            pltpu.VMEM((_CH, d), jnp.float32),
            pltpu.VMEM((_CH, d), jnp.float32),
            pltpu.VMEM((_CH, d), jnp.float32),
            pltpu.VMEM((_CH, d), jnp.float32),
            pltpu.VMEM_SHARED((n_pad, d), jnp.float32),
            pltpu.SemaphoreType.DMA,
            pltpu.SemaphoreType.DMA,
            pltpu.SemaphoreType.DMA,
            pltpu.SemaphoreType.DMA,
            pltpu.SemaphoreType.DMA,
            pltpu.SemaphoreType.DMA,
            pltpu.SemaphoreType.DMA,
            pltpu.SemaphoreType.DMA,
        ],
        compiler_params=pltpu.CompilerParams(use_tc_tiling_on_sc=False),
    )
    def k(table_h, ei_h, eip_h, out_h, ij_v, rows0_v, rows1_v, rows2_v,
          rows3_v, acc_sh, gsem0, gsem1, gsem2, gsem3, ssem0, ssem1, ssem2,
          ssem3):
        cid = lax.axis_index("c")
        sid = lax.axis_index("s")
        wid = sid * nc + cid
        r0 = sid * rpt

        @pl.when(wid < nw - 1)
        def _():
            pltpu.sync_copy(ei_h.at[:, pl.ds(wid * cpt, cpt)], ij_v)

        @pl.when(wid == nw - 1)
        def _():
            pltpu.sync_copy(ei_h.at[:, pl.ds((nw - 1) * cpt, split)],
                            ij_v.at[:, pl.ds(0, split)])
            pltpu.sync_copy(eip_h, ij_v.at[:, pl.ds(split, pad_chunks)])

        for b in range(rpt // _ROWPAD):
            if b < rpt_last // _ROWPAD:
                pltpu.sync_copy(table_h.at[pl.ds(zrow0, _ROWPAD)],
                                acc_sh.at[pl.ds(r0 + b * _ROWPAD, _ROWPAD)])
            else:
                @pl.when(sid < ns - 1)
                def _(b=b):
                    pltpu.sync_copy(table_h.at[pl.ds(zrow0, _ROWPAD)],
                                    acc_sh.at[pl.ds(r0 + b * _ROWPAD,
                                                    _ROWPAD)])

        tail = rpt_last % _ROWPAD
        if tail:
            @pl.when(sid == ns - 1)
            def _():
                pltpu.sync_copy(
                    table_h.at[pl.ds(zrow0, tail)],
                    acc_sh.at[pl.ds(r0 + (rpt_last // _ROWPAD) * _ROWPAD,
                                    tail)])

        plsc.subcore_barrier()

        rows = [rows0_v, rows1_v, rows2_v, rows3_v]
        gsems = [gsem0, gsem1, gsem2, gsem3]
        ssems = [ssem0, ssem1, ssem2, ssem3]

        def gather(j, t):
            pltpu.async_copy(table_h.at[ij_v.at[0, j]], rows[t], gsems[t])

        def gather_wait(j, t):
            pltpu.make_async_copy(table_h.at[ij_v.at[0, j]], rows[t],
                                  gsems[t]).wait()

        def scatter(j, t):
            pltpu.async_copy(rows[t], acc_sh.at[ij_v.at[1, j]], ssems[t],
                             add=True)

        def scatter_wait(j, t):
            pltpu.make_async_copy(rows[t], acc_sh.at[ij_v.at[1, j]],
                                  ssems[t]).wait()

        def quad(q, carry):
            j0 = 4 * q
            for t in range(4):
                @pl.when(q > 0)
                def _(j=j0 + t, t=t):
                    scatter_wait(j - 4, t)

                gather(j0 + t, t)
            for t in range(4):
                gather_wait(j0 + t, t)
                scatter(j0 + t, t)
            return carry

        lax.fori_loop(0, cpt // 4, quad, 0)
        for t in range(4):
            scatter_wait(cpt - 4 + t, t)
        plsc.subcore_barrier()

        @pl.when(sid < ns - 1)
        def _():
            pltpu.sync_copy(acc_sh.at[pl.ds(r0, rpt)],
                            out_h.at[cid, pl.ds(r0, rpt), pl.ds(0, d)])

        @pl.when(sid == ns - 1)
        def _():
            pltpu.sync_copy(acc_sh.at[pl.ds((ns - 1) * rpt, rpt_last)],
                            out_h.at[cid, pl.ds((ns - 1) * rpt, rpt_last),
                                     pl.ds(0, d)])

    return k(table, ei_real, ei_pad)


def _tc_proj1(x, w_self, w_neigh_pad):
    n, _ = x.shape
    d = w_self.shape[1]
    tw = w_neigh_pad.shape[1]

    def body(x_ref, ws_ref, wn_ref, s_ref, t_ref):
        xv = x_ref[...]
        s_ref[...] = jnp.dot(xv, ws_ref[...], preferred_element_type=jnp.float32)
        t = jnp.dot(xv, wn_ref[...], preferred_element_type=jnp.float32)
        col = lax.broadcasted_iota(jnp.int32, (n, tw), 1)
        t_ref[:n] = t + jnp.where(col == d, 1.0, 0.0)
        t_ref[n:] = jnp.zeros((_ROWPAD, tw), jnp.float32)

    return pl.pallas_call(
        body,
        out_shape=(jax.ShapeDtypeStruct((n, d), jnp.float32),
                   jax.ShapeDtypeStruct((n + _ROWPAD, tw), jnp.float32)),
    )(x, w_self, w_neigh_pad)


def _tc_mid(s1x, acc1, b1, w_self2, w_neigh2_pad):
    n, d = s1x.shape
    tw = w_neigh2_pad.shape[1]

    def body(s_ref, a_ref, b_ref, ws_ref, wn_ref, s2_ref, t2_ref, dinv_ref):
        dp = d + 1
        acc = a_ref[0, :n, :dp] + a_ref[1, :n, :dp]
        deg = acc[:, d:d + 1]
        dinv = 1.0 / jnp.maximum(deg, 1.0)
        h = jnp.maximum(s_ref[...] + acc[:, :d] * dinv + b_ref[...], 0.0)
        s2_ref[...] = jnp.dot(h, ws_ref[...], preferred_element_type=jnp.float32)
        t2_ref[:n] = jnp.dot(h, wn_ref[...], preferred_element_type=jnp.float32)
        t2_ref[n:] = jnp.zeros((_ROWPAD, tw), jnp.float32)
        dinv_ref[...] = dinv

    return pl.pallas_call(
        body,
        out_shape=(jax.ShapeDtypeStruct((n, d), jnp.float32),
                   jax.ShapeDtypeStruct((n + _ROWPAD, tw), jnp.float32),
                   jax.ShapeDtypeStruct((n, 1), jnp.float32)),
    )(s1x, acc1, b1.reshape(1, d), w_self2, w_neigh2_pad)


def _tc_decoder(s2h, acc2, dinv, b2, block_rows=400):
    n, d = s2h.shape
    n_pad2 = acc2.shape[1]

    def body(s_ref, a_ref, dinv_ref, b_ref, z_ref, o_ref):
        i = pl.program_id(0)

        @pl.when(i == 0)
        def _():
            acc = a_ref[0, :n, :d] + a_ref[1, :n, :d]
            z_ref[...] = s_ref[...] + acc * dinv_ref[...] + b_ref[...]

        zb = z_ref[pl.ds(i * block_rows, block_rows), :]
        logits = lax.dot_general(zb, z_ref[...], (((1,), (1,)), ((), ())),
                                 preferred_element_type=jnp.float32)
        o_ref[...] = 0.5 * jnp.tanh(0.5 * logits) + 0.5

    grid = (n // block_rows,)
    return pl.pallas_call(
        body,
        grid=grid,
        in_specs=[
            pl.BlockSpec((n, d), lambda i: (0, 0)),
            pl.BlockSpec((2, n_pad2, 128), lambda i: (0, 0, 0)),
            pl.BlockSpec((n, 1), lambda i: (0, 0)),
            pl.BlockSpec((1, d), lambda i: (0, 0)),
        ],
        out_specs=(
            pl.BlockSpec((n, d), lambda i: (0, 0)),
            pl.BlockSpec((block_rows, n), lambda i: (i, 0)),
        ),
        out_shape=(
            jax.ShapeDtypeStruct((n, d), jnp.float32),
            jax.ShapeDtypeStruct((n, n), jnp.float32),
        ),
    )(s2h, acc2, dinv, b2.reshape(1, d))


def kernel(features, edge_index, W_self1, W_neigh1, b1, W_self2, W_neigh2, b2):
    n, _ = features.shape
    d = W_self1.shape[1]
    dp = 80
    e = edge_index.shape[1]

    info = plsc.get_sparse_core_info()
    nw = info.num_cores * info.num_subcores
    assert e % _CH == 0
    n_chunks_real = e // _CH
    unit = 2 * nw
    n_chunks_pad = (n_chunks_real + unit - 1) // unit * unit
    pad_len = (n_chunks_pad - n_chunks_real) * _CH
    pad_iota = jnp.arange(pad_len, dtype=jnp.int32)
    pad_src = (n + pad_iota % _ROWPAD).reshape(1, -1, _CH)
    pad_dst = (pad_iota % n).reshape(1, -1, _CH)
    ei_pad = jnp.concatenate([pad_src, pad_dst], axis=0)
    ei_real = edge_index.reshape(2, n_chunks_real, _CH)

    w_neigh1_pad = jnp.pad(W_neigh1, ((0, 0), (0, dp - d)))
    s1x, table1 = _tc_proj1(features, W_self1, w_neigh1_pad)

    acc1 = _sc_segment_sum(table1, ei_real, ei_pad)
    s2h, table2, dinv = _tc_mid(s1x, acc1, b1, W_self2, W_neigh2)

    acc2 = _sc_segment_sum(table2, ei_real, ei_pad)
    z, adj = _tc_decoder(s2h, acc2, dinv, b2)
    return (z, adj)


# speedup vs baseline: 1.0461x; 1.0461x over previous
import functools

import jax
import jax.numpy as jnp
from jax import lax
from jax.experimental import pallas as pl
from jax.experimental.pallas import tpu as pltpu
from jax.experimental.pallas import tpu_sc as plsc

_CH = 128
_ROWPAD = 128


def _sc_segment_sum(table, ei_real, ei_pad):
    n_pad, d = table.shape
    n_chunks_real = ei_real.shape[1]
    pad_chunks = ei_pad.shape[1]
    n_chunks = n_chunks_real + pad_chunks
    info = plsc.get_sparse_core_info()
    nc, ns = info.num_cores, info.num_subcores
    nw = nc * ns
    cpt = n_chunks // nw
    assert cpt * nw == n_chunks and cpt % 4 == 0
    assert pad_chunks < cpt
    split = n_chunks_real - (nw - 1) * cpt
    zrow0 = n_pad - _ROWPAD
    rpt = ((n_pad + ns - 1) // ns + 7) // 8 * 8
    rpt_last = n_pad - rpt * (ns - 1)
    assert rpt_last > 0 and rpt_last % 8 == 0

    mesh = plsc.VectorSubcoreMesh(core_axis_name="c", subcore_axis_name="s")

    @functools.partial(
        pl.kernel,
        out_type=jax.ShapeDtypeStruct((nc, n_pad, 128), jnp.float32),
        mesh=mesh,
        scratch_types=[
            pltpu.VMEM((2, cpt, _CH), jnp.int32),
            pltpu.VMEM((_CH, d), jnp.float32),
            pltpu.VMEM((_CH, d), jnp.float32),
            pltpu.VMEM((_CH, d), jnp.float32),
            pltpu.VMEM((_CH, d), jnp.float32),
            pltpu.VMEM_SHARED((n_pad, d), jnp.float32),
            pltpu.SemaphoreType.DMA,
            pltpu.SemaphoreType.DMA,
            pltpu.SemaphoreType.DMA,
            pltpu.SemaphoreType.DMA,
            pltpu.SemaphoreType.DMA,
            pltpu.SemaphoreType.DMA,
            pltpu.SemaphoreType.DMA,
            pltpu.SemaphoreType.DMA,
        ],
        compiler_params=pltpu.CompilerParams(use_tc_tiling_on_sc=False),
    )
    def k(table_h, ei_h, eip_h, zeros_h, out_h, ij_v, rows0_v, rows1_v,
          rows2_v, rows3_v, acc_sh, gsem0, gsem1, gsem2, gsem3, ssem0, ssem1,
          ssem2, ssem3):
        cid = lax.axis_index("c")
        sid = lax.axis_index("s")
        wid = sid * nc + cid
        r0 = sid * rpt

        @pl.when(wid < nw - 1)
        def _():
            pltpu.sync_copy(ei_h.at[:, pl.ds(wid * cpt, cpt)], ij_v)

        @pl.when(wid == nw - 1)
        def _():
            pltpu.sync_copy(ei_h.at[:, pl.ds((nw - 1) * cpt, split)],
                            ij_v.at[:, pl.ds(0, split)])
            pltpu.sync_copy(eip_h, ij_v.at[:, pl.ds(split, pad_chunks)])

        @pl.when(sid < ns - 1)
        def _():
            pltpu.sync_copy(zeros_h.at[pl.ds(r0, rpt)],
                            acc_sh.at[pl.ds(r0, rpt)])

        @pl.when(sid == ns - 1)
        def _():
            pltpu.sync_copy(zeros_h.at[pl.ds((ns - 1) * rpt, rpt_last)],
                            acc_sh.at[pl.ds((ns - 1) * rpt, rpt_last)])

        plsc.subcore_barrier()

        rows = [rows0_v, rows1_v, rows2_v, rows3_v]
        gsems = [gsem0, gsem1, gsem2, gsem3]
        ssems = [ssem0, ssem1, ssem2, ssem3]

        def gather(j, t):
            pltpu.async_copy(table_h.at[ij_v.at[0, j]], rows[t], gsems[t])

        def gather_wait(j, t):
            pltpu.make_async_copy(table_h.at[ij_v.at[0, j]], rows[t],
                                  gsems[t]).wait()

        def scatter(j, t):
            pltpu.async_copy(rows[t], acc_sh.at[ij_v.at[1, j]], ssems[t],
                             add=True)

        def scatter_wait(j, t):
            pltpu.make_async_copy(rows[t], acc_sh.at[ij_v.at[1, j]],
                                  ssems[t]).wait()

        def quad(q, carry):
            j0 = 4 * q
            for t in range(4):
                @pl.when(q > 0)
                def _(j=j0 + t, t=t):
                    scatter_wait(j - 4, t)

                gather(j0 + t, t)
            for t in range(4):
                gather_wait(j0 + t, t)
                scatter(j0 + t, t)
            return carry

        lax.fori_loop(0, cpt // 4, quad, 0)
        for t in range(4):
            scatter_wait(cpt - 4 + t, t)
        plsc.subcore_barrier()

        @pl.when(sid < ns - 1)
        def _():
            pltpu.sync_copy(acc_sh.at[pl.ds(r0, rpt)],
                            out_h.at[cid, pl.ds(r0, rpt), pl.ds(0, d)])

        @pl.when(sid == ns - 1)
        def _():
            pltpu.sync_copy(acc_sh.at[pl.ds((ns - 1) * rpt, rpt_last)],
                            out_h.at[cid, pl.ds((ns - 1) * rpt, rpt_last),
                                     pl.ds(0, d)])

    return k(table, ei_real, ei_pad, jnp.zeros((n_pad, d), jnp.float32))


def _tc_proj1(x, w_self, w_neigh_pad):
    n, _ = x.shape
    d = w_self.shape[1]
    tw = w_neigh_pad.shape[1]

    def body(x_ref, ws_ref, wn_ref, s_ref, t_ref):
        xv = x_ref[...]
        s_ref[...] = jnp.dot(xv, ws_ref[...], preferred_element_type=jnp.float32)
        t = jnp.dot(xv, wn_ref[...], preferred_element_type=jnp.float32)
        col = lax.broadcasted_iota(jnp.int32, (n, tw), 1)
        t_ref[:n] = t + jnp.where(col == d, 1.0, 0.0)
        t_ref[n:] = jnp.zeros((_ROWPAD, tw), jnp.float32)

    return pl.pallas_call(
        body,
        out_shape=(jax.ShapeDtypeStruct((n, d), jnp.float32),
                   jax.ShapeDtypeStruct((n + _ROWPAD, tw), jnp.float32)),
    )(x, w_self, w_neigh_pad)


def _tc_mid(s1x, acc1, b1, w_self2, w_neigh2_pad):
    n, d = s1x.shape
    tw = w_neigh2_pad.shape[1]

    def body(s_ref, a_ref, b_ref, ws_ref, wn_ref, s2_ref, t2_ref, dinv_ref):
        dp = d + 1
        acc = a_ref[0, :n, :dp] + a_ref[1, :n, :dp]
        deg = acc[:, d:d + 1]
        dinv = 1.0 / jnp.maximum(deg, 1.0)
        h = jnp.maximum(s_ref[...] + acc[:, :d] * dinv + b_ref[...], 0.0)
        s2_ref[...] = jnp.dot(h, ws_ref[...], preferred_element_type=jnp.float32)
        t2_ref[:n] = jnp.dot(h, wn_ref[...], preferred_element_type=jnp.float32)
        t2_ref[n:] = jnp.zeros((_ROWPAD, tw), jnp.float32)
        dinv_ref[...] = dinv

    return pl.pallas_call(
        body,
        out_shape=(jax.ShapeDtypeStruct((n, d), jnp.float32),
                   jax.ShapeDtypeStruct((n + _ROWPAD, tw), jnp.float32),
                   jax.ShapeDtypeStruct((n, 1), jnp.float32)),
    )(s1x, acc1, b1.reshape(1, d), w_self2, w_neigh2_pad)


def _tc_decoder(s2h, acc2, dinv, b2, block_rows=400):
    n, d = s2h.shape
    n_pad2 = acc2.shape[1]

    def body(s_ref, a_ref, dinv_ref, b_ref, z_ref, o_ref):
        i = pl.program_id(0)

        @pl.when(i == 0)
        def _():
            acc = a_ref[0, :n, :d] + a_ref[1, :n, :d]
            z_ref[...] = s_ref[...] + acc * dinv_ref[...] + b_ref[...]

        zb = z_ref[pl.ds(i * block_rows, block_rows), :]
        logits = lax.dot_general(zb, z_ref[...], (((1,), (1,)), ((), ())),
                                 preferred_element_type=jnp.float32)
        o_ref[...] = 0.5 * jnp.tanh(0.5 * logits) + 0.5

    grid = (n // block_rows,)
    return pl.pallas_call(
        body,
        grid=grid,
        in_specs=[
            pl.BlockSpec((n, d), lambda i: (0, 0)),
            pl.BlockSpec((2, n_pad2, 128), lambda i: (0, 0, 0)),
            pl.BlockSpec((n, 1), lambda i: (0, 0)),
            pl.BlockSpec((1, d), lambda i: (0, 0)),
        ],
        out_specs=(
            pl.BlockSpec((n, d), lambda i: (0, 0)),
            pl.BlockSpec((block_rows, n), lambda i: (i, 0)),
        ),
        out_shape=(
            jax.ShapeDtypeStruct((n, d), jnp.float32),
            jax.ShapeDtypeStruct((n, n), jnp.float32),
        ),
    )(s2h, acc2, dinv, b2.reshape(1, d))


def kernel(features, edge_index, W_self1, W_neigh1, b1, W_self2, W_neigh2, b2):
    n, _ = features.shape
    d = W_self1.shape[1]
    dp = 80
    e = edge_index.shape[1]

    info = plsc.get_sparse_core_info()
    nw = info.num_cores * info.num_subcores
    assert e % _CH == 0
    n_chunks_real = e // _CH
    unit = 2 * nw
    n_chunks_pad = (n_chunks_real + unit - 1) // unit * unit
    pad_len = (n_chunks_pad - n_chunks_real) * _CH
    pad_iota = jnp.arange(pad_len, dtype=jnp.int32)
    pad_src = (n + pad_iota % _ROWPAD).reshape(1, -1, _CH)
    pad_dst = (pad_iota % n).reshape(1, -1, _CH)
    ei_pad = jnp.concatenate([pad_src, pad_dst], axis=0)
    ei_real = edge_index.reshape(2, n_chunks_real, _CH)

    w_neigh1_pad = jnp.pad(W_neigh1, ((0, 0), (0, dp - d)))
    s1x, table1 = _tc_proj1(features, W_self1, w_neigh1_pad)

    acc1 = _sc_segment_sum(table1, ei_real, ei_pad)
    s2h, table2, dinv = _tc_mid(s1x, acc1, b1, W_self2, W_neigh2)

    acc2 = _sc_segment_sum(table2, ei_real, ei_pad)
    z, adj = _tc_decoder(s2h, acc2, dinv, b2)
    return (z, adj)
